# Initial kernel scaffold; baseline (speedup 1.0000x reference)
import jax
import jax.numpy as jnp
from jax.experimental import pallas as pl


def kernel(h, pe, x, t, context, edges, edge_index, edge_attr, batch, params):
    raise NotImplementedError("write your pallas kernel here")


# speedup vs baseline: 2.0554x; 2.0554x over previous
import functools
import math

import jax
import jax.numpy as jnp
from jax import lax
from jax.experimental import pallas as pl
from jax.experimental.pallas import tpu as pltpu
from jax.experimental.pallas import tpu_sc as plsc

F32 = jnp.float32
I32 = jnp.int32

NW = 32
CH = 512
IDX_ROWS = CH // 128


def _silu(v):
    return v / (1.0 + jnp.exp(-v))


def _stats_body(pe_ref, out_ref):
    pe = pe_ref[...]
    n = pe.shape[0]
    mean = jnp.sum(pe, axis=0, keepdims=True) / n
    var = jnp.sum((pe - mean) ** 2, axis=0, keepdims=True) / n
    out_ref[0:1, :] = mean
    out_ref[1:2, :] = var


def _node_pre_body(h_ref, pe_ref, t8_ref, ctx_ref, stats_ref, node_W, node_b,
                   pe_W, pe_b, ctx_W, ctx_b, bn_w, bn_b, WeT1, WeT2,
                   hc_ref, tp1_ref, tp2_ref):
    mean = stats_ref[0:1, :]
    var = stats_ref[1:2, :]
    pe_n = (pe_ref[...] - mean) * lax.rsqrt(var + 1e-5) * bn_w[...] + bn_b[...]
    freqs = jnp.exp(-(math.log(10000.0) / 4.0)
                    * lax.broadcasted_iota(I32, (1, 4), 1).astype(F32))
    args = t8_ref[:, 0:4] * freqs
    emb = jnp.concatenate([jnp.cos(args), jnp.sin(args)], axis=1)
    hc_ref[...] = jnp.concatenate(
        [h_ref[...] @ node_W[...] + node_b[...],
         pe_n @ pe_W[...] + pe_b[...],
         emb,
         ctx_ref[...] @ ctx_W[...] + ctx_b[...]], axis=1)
    tp1_ref[...] = emb @ WeT1[...]
    tp2_ref[...] = emb @ WeT2[...]


def _node_lin_body(hc_ref, inW, inb, Wi, Wj, h1_ref, p_ref, q_ref):
    h1 = hc_ref[...] @ inW[...] + inb[...]
    h1_ref[...] = h1
    p_ref[...] = h1 @ Wi[...]
    q_ref[...] = h1 @ Wj[...]


def _edge_mlp_body(ga_ref, gb_ref, xr_ref, xc_ref, ea_ref, tp_ref,
                   wr, WeA, be, e2_W, e2_b, m2_ref):
    d = xr_ref[...] - xc_ref[...]
    radial = jnp.sum(d * d, axis=1, keepdims=True)
    be_blk = ga_ref.shape[0]
    rsel = (lax.broadcasted_iota(I32, (be_blk, 128), 0) // 16
            == lax.broadcasted_iota(I32, (be_blk, 128), 1)).astype(F32)
    te = rsel @ tp_ref[...]
    pre = (ga_ref[...] + gb_ref[...] + radial * wr[...]
           + ea_ref[...] @ WeA[...] + te + be[...])
    m = _silu(pre)
    m2_ref[...] = _silu(m @ e2_W[...] + e2_b[...])


def _node_upd_body(h1_ref, agg_ref, n1a, n1b, n1bias, n2_W, n2_b,
                   out_W, out_b, hc_ref):
    h1 = h1_ref[...]
    z = _silu(h1 @ n1a[...] + agg_ref[...] @ n1b[...] + n1bias[...])
    h1b = h1 + z @ n2_W[...] + n2_b[...]
    hc_ref[...] = h1b @ out_W[...] + out_b[...]


def _pool_head_body(hc_ref, b8_ref, m1_W, m1_b, m2_W, m2_b, m3_W, m3_b,
                    head_ref, acc_ref):
    i = pl.program_id(0)

    @pl.when(i == 0)
    def _():
        acc_ref[...] = jnp.zeros_like(acc_ref)

    bn = hc_ref.shape[0]
    oh = (b8_ref[:, 0:1] == lax.broadcasted_iota(I32, (bn, 512), 1)).astype(F32)
    acc_ref[...] += lax.dot_general(oh, hc_ref[...],
                                    (((0,), (0,)), ((), ())),
                                    preferred_element_type=F32)

    @pl.when(i == pl.num_programs(0) - 1)
    def _():
        pooled = acc_ref[...]
        z = jnp.maximum(pooled @ m1_W[...] + m1_b[...], 0.0)
        z = jnp.maximum(z @ m2_W[...] + m2_b[...], 0.0)
        head_ref[...] = z @ m3_W[...] + m3_b[...]


def _full(shape):
    return pl.BlockSpec(shape, lambda i: (0,) * len(shape))


def _blk(shape):
    return pl.BlockSpec(shape, lambda i: (i,) + (0,) * (len(shape) - 1))


def _make_gather(n_rows, width, e_pad):
    per_w = e_pad // NW
    n_ch = per_w // CH
    mesh = plsc.VectorSubcoreMesh(core_axis_name="c", subcore_axis_name="s")

    @functools.partial(
        pl.kernel, mesh=mesh,
        compiler_params=pltpu.CompilerParams(use_tc_tiling_on_sc=False),
        out_type=[jax.ShapeDtypeStruct((e_pad, width), F32),
                  jax.ShapeDtypeStruct((e_pad, width), F32)],
        scratch_types=[pltpu.VMEM((IDX_ROWS, 128), I32),
                       pltpu.VMEM((IDX_ROWS, 128), I32),
                       pltpu.VMEM((CH, width), F32),
                       pltpu.VMEM((CH, width), F32),
                       pltpu.SemaphoreType.DMA,
                       pltpu.SemaphoreType.DMA],
    )
    def gather(tabA, tabB, idxA2d, idxB2d, outA, outB,
               ia, ib, bufa, bufb, sema, semb):
        wid = lax.axis_index("s") * 2 + lax.axis_index("c")
        rows0 = wid * (per_w // 128)

        def body(i, carry):
            rb = rows0 + i * IDX_ROWS
            pltpu.sync_copy(idxA2d.at[pl.ds(rb, IDX_ROWS)], ia)
            pltpu.sync_copy(idxB2d.at[pl.ds(rb, IDX_ROWS)], ib)
            hs = []
            for j in range(IDX_ROWS):
                hs.append(pltpu.async_copy(
                    tabA.at[ia.at[j]], bufa.at[pl.ds(j * 128, 128)], sema))
                hs.append(pltpu.async_copy(
                    tabB.at[ib.at[j]], bufb.at[pl.ds(j * 128, 128)], semb))
            for hnd in hs:
                hnd.wait()
            eb = wid * per_w + i * CH
            pltpu.sync_copy(bufa, outA.at[pl.ds(eb, CH)])
            pltpu.sync_copy(bufb, outB.at[pl.ds(eb, CH)])
            return carry

        lax.fori_loop(0, n_ch, body, 0)

    return gather


def _make_scatter(n_nodes, e_pad, n_trash):
    n_acc = n_nodes + n_trash
    per_tile_rows = n_acc // 16
    drain_rows = n_nodes // 16
    per_t = e_pad // 16
    n_ch = per_t // CH
    mesh = plsc.VectorSubcoreMesh(core_axis_name="c", subcore_axis_name="s")

    @functools.partial(
        pl.kernel, mesh=mesh,
        compiler_params=pltpu.CompilerParams(use_tc_tiling_on_sc=False),
        out_type=jax.ShapeDtypeStruct((2, n_nodes, 32), F32),
        scratch_types=[pltpu.VMEM((IDX_ROWS, 128), I32),
                       pltpu.VMEM((CH, 32), F32),
                       pltpu.VMEM_SHARED((n_acc, 32), F32)],
    )
    def scatter(m2, idx2d, zeros_hbm, agg, idx, upd, acc):
        c = lax.axis_index("c")
        s = lax.axis_index("s")
        pltpu.sync_copy(zeros_hbm,
                        acc.at[pl.ds(s * per_tile_rows, per_tile_rows)])
        plsc.subcore_barrier()

        def body(i, carry):
            eb = s * per_t + i * CH
            rb = eb // 128
            pltpu.sync_copy(idx2d.at[pl.ds(rb, IDX_ROWS)], idx)
            pltpu.sync_copy(m2.at[pl.ds(eb, CH), c], upd)
            for j in range(IDX_ROWS):
                pltpu.sync_copy(upd.at[pl.ds(j * 128, 128)],
                                acc.at[idx.at[j]], add=True)
            return carry

        lax.fori_loop(0, n_ch, body, 0)
        plsc.subcore_barrier()
        pltpu.sync_copy(acc.at[pl.ds(s * drain_rows, drain_rows)],
                        agg.at[c, pl.ds(s * drain_rows, drain_rows)])

    return scatter


def kernel(h, pe, x, t, context, edges, edge_index, edge_attr, batch, params):
    p = params
    N = h.shape[0]
    E = edge_attr.shape[0]
    G = 512
    BN = N // 10
    BE = 2048
    e_pad = ((E + NW * CH - 1) // (NW * CH)) * (NW * CH)
    n_trash = 48

    rows = edges[0]
    cols = edges[1]
    pad = e_pad - E
    rows_g = jnp.concatenate([rows, jnp.zeros((pad,), I32)]).reshape(-1, 128)
    cols_g = jnp.concatenate([cols, jnp.zeros((pad,), I32)]).reshape(-1, 128)
    rows_s = jnp.concatenate(
        [rows, N + (jnp.arange(pad, dtype=I32) % n_trash)]).reshape(-1, 128)
    ea_pad = jnp.concatenate([edge_attr, jnp.zeros((pad, 4), F32)])
    x_pad = jnp.pad(x, ((0, 0), (0, 13)))
    t8 = jnp.tile(t.reshape(N, 1), (1, 8))
    b8 = jnp.tile(batch.reshape(N, 1), (1, 8))
    zeros_tile = jnp.zeros(((N + n_trash) // 16, 32), F32)

    convs = p['convs']
    folded = []
    for c in convs:
        folded.append(dict(
            Wi=c['e1_W'][0:64], Wj=c['e1_W'][64:128],
            wr=c['e1_W'][128:129],
            WeA=p['edge_W'] @ c['e1_W'][129:185],
            WeT=c['e1_W'][185:193],
            be=(c['e1_b'] + p['edge_b'] @ c['e1_W'][129:185]).reshape(1, 64),
            inW=c['in_W'], inb=c['in_b'].reshape(1, 64),
            n1a=c['n1_W'][:64], n1b=c['n1_W'][64:],
            n1bias=c['n1_b'].reshape(1, 64),
            n2W=c['n2_W'], n2b=c['n2_b'].reshape(1, 64),
            outW=c['out_W'], outb=c['out_b'].reshape(1, 64),
            e2W=c['e2_W'], e2b=c['e2_b'].reshape(1, 64)))

    stats = pl.pallas_call(
        _stats_body,
        grid=(1,),
        in_specs=[_full((N, 20))],
        out_specs=_full((8, 20)),
        out_shape=jax.ShapeDtypeStruct((8, 20), F32),
    )(pe)

    hc, tp1, tp2 = pl.pallas_call(
        _node_pre_body,
        grid=(10,),
        in_specs=[_blk((BN, 64)), _blk((BN, 20)), _blk((BN, 8)),
                  _blk((BN, 64)), _full((8, 20)),
                  _full((64, 32)), _full((1, 32)),
                  _full((20, 8)), _full((1, 8)),
                  _full((64, 16)), _full((1, 16)),
                  _full((1, 20)), _full((1, 20)),
                  _full((8, 64)), _full((8, 64))],
        out_specs=[_blk((BN, 64)), _blk((BN, 64)), _blk((BN, 64))],
        out_shape=[jax.ShapeDtypeStruct((N, 64), F32)] * 3,
    )(h, pe, t8, context, stats,
      p['node_W'], p['node_b'].reshape(1, 32),
      p['pe_W'], p['pe_b'].reshape(1, 8),
      p['ctx_W'], p['ctx_b'].reshape(1, 16),
      p['bn_w'].reshape(1, 20), p['bn_b'].reshape(1, 20),
      folded[0]['WeT'], folded[1]['WeT'])

    xr, xc = _make_gather(N, 16, e_pad)(x_pad, x_pad, rows_g, cols_g)

    gather64 = _make_gather(N, 64, e_pad)
    scatter = _make_scatter(N, e_pad, n_trash)
    tps = [tp1, tp2]

    for li, fc in enumerate(folded):
        h1, P, Q = pl.pallas_call(
            _node_lin_body,
            grid=(10,),
            in_specs=[_blk((BN, 64)), _full((64, 64)), _full((1, 64)),
                      _full((64, 64)), _full((64, 64))],
            out_specs=[_blk((BN, 64))] * 3,
            out_shape=[jax.ShapeDtypeStruct((N, 64), F32)] * 3,
        )(hc, fc['inW'], fc['inb'], fc['Wi'], fc['Wj'])

        ga, gb = gather64(P, Q, rows_g, cols_g)

        tp_full = jnp.concatenate(
            [tps[li], tps[li][0:1],
             jnp.zeros((e_pad // 16 - N - 1, 64), F32)])

        m2 = pl.pallas_call(
            _edge_mlp_body,
            grid=(e_pad // BE,),
            in_specs=[_blk((BE, 64)), _blk((BE, 64)),
                      _blk((BE, 16)), _blk((BE, 16)), _blk((BE, 4)),
                      _blk((BE // 16, 64)),
                      _full((1, 64)), _full((4, 64)), _full((1, 64)),
                      _full((64, 64)), _full((1, 64))],
            out_specs=_blk((BE, 64)),
            out_shape=jax.ShapeDtypeStruct((e_pad, 64), F32),
        )(ga, gb, xr, xc, ea_pad, tp_full,
          fc['wr'], fc['WeA'], fc['be'], fc['e2W'], fc['e2b'])

        aggs = scatter(m2.reshape(e_pad, 2, 32), rows_s, zeros_tile)
        agg = jnp.concatenate([aggs[0], aggs[1]], axis=1)

        hc = pl.pallas_call(
            _node_upd_body,
            grid=(10,),
            in_specs=[_blk((BN, 64)), _blk((BN, 64)),
                      _full((64, 64)), _full((64, 64)), _full((1, 64)),
                      _full((64, 64)), _full((1, 64)),
                      _full((64, 64)), _full((1, 64))],
            out_specs=_blk((BN, 64)),
            out_shape=jax.ShapeDtypeStruct((N, 64), F32),
        )(h1, agg, fc['n1a'], fc['n1b'], fc['n1bias'],
          fc['n2W'], fc['n2b'], fc['outW'], fc['outb'])

    out = pl.pallas_call(
        _pool_head_body,
        grid=(10,),
        in_specs=[_blk((BN, 64)), _blk((BN, 8)),
                  _full((64, 32)), _full((1, 32)),
                  _full((32, 16)), _full((1, 16)),
                  _full((16, 1)), _full((1, 1))],
        out_specs=_full((G, 1)),
        out_shape=jax.ShapeDtypeStruct((G, 1), F32),
        scratch_shapes=[pltpu.VMEM((G, 64), F32)],
    )(hc, b8,
      p['m1_W'], p['m1_b'].reshape(1, 32),
      p['m2_W'], p['m2_b'].reshape(1, 16),
      p['m3_W'], p['m3_b'].reshape(1, 1))
    return out


# speedup vs baseline: 2.7715x; 1.3484x over previous
import functools
import math

import jax
import jax.numpy as jnp
from jax import lax
from jax.experimental import pallas as pl
from jax.experimental.pallas import tpu as pltpu
from jax.experimental.pallas import tpu_sc as plsc

F32 = jnp.float32
I32 = jnp.int32

NW = 32
GRP = 1024
CHG = 128
CHS = 512
N_TRASH = 1200


def _silu(v):
    return v / (1.0 + jnp.exp(-v))


def _stats_body(pe_ref, out_ref):
    pe = pe_ref[...]
    n = pe.shape[0]
    mean = jnp.sum(pe, axis=0, keepdims=True) / n
    var = jnp.sum((pe - mean) ** 2, axis=0, keepdims=True) / n
    out_ref[0:1, :] = mean
    out_ref[1:2, :] = var


def _node_pre_body(h_ref, pe_ref, t8_ref, ctx_ref, stats_ref, node_W, node_b,
                   pe_W, pe_b, ctx_W, ctx_b, bn_w, bn_b, WeT1, WeT2,
                   hc_ref, tp1_ref, tp2_ref):
    mean = stats_ref[0:1, :]
    var = stats_ref[1:2, :]
    pe_n = (pe_ref[...] - mean) * lax.rsqrt(var + 1e-5) * bn_w[...] + bn_b[...]
    freqs = jnp.exp(-(math.log(10000.0) / 4.0)
                    * lax.broadcasted_iota(I32, (1, 4), 1).astype(F32))
    args = t8_ref[:, 0:4] * freqs
    emb = jnp.concatenate([jnp.cos(args), jnp.sin(args)], axis=1)
    hc_ref[...] = jnp.concatenate(
        [h_ref[...] @ node_W[...] + node_b[...],
         pe_n @ pe_W[...] + pe_b[...],
         emb,
         ctx_ref[...] @ ctx_W[...] + ctx_b[...]], axis=1)
    tp1_ref[...] = emb @ WeT1[...]
    tp2_ref[...] = emb @ WeT2[...]


def _node_lin_body(hc_ref, xp_ref, inW, inb, Wi, Wj, h1_ref, ta_ref, tb_ref):
    h1 = hc_ref[...] @ inW[...] + inb[...]
    h1_ref[...] = h1
    xp = xp_ref[...]
    zer = jnp.zeros((h1.shape[0], 48), F32)
    ta_ref[...] = jnp.concatenate([h1 @ Wi[...], xp, zer], axis=1)
    tb_ref[...] = jnp.concatenate([h1 @ Wj[...], -xp, zer], axis=1)


def _edge_mlp_body(s_ref, ea_ref, tp_ref, wr, WeA, be, e2_W, e2_b,
                   m2a_ref, m2b_ref):
    s = s_ref[...]
    dx = s[:, 64:80]
    radial = jnp.sum(dx * dx, axis=1, keepdims=True)
    be_blk = s.shape[0]
    rsel = (lax.broadcasted_iota(I32, (be_blk, 128), 0) // 16
            == lax.broadcasted_iota(I32, (be_blk, 128), 1)).astype(F32)
    te = rsel @ tp_ref[...]
    pre = s[:, 0:64] + radial * wr[...] + ea_ref[...] @ WeA[...] + te + be[...]
    m = _silu(pre)
    m2 = _silu(m @ e2_W[...] + e2_b[...])
    qb = be_blk // 4
    m2a_ref[...] = jnp.concatenate(
        [m2[k * qb:(k + 1) * qb, 0:32] for k in range(4)], axis=1)
    m2b_ref[...] = jnp.concatenate(
        [m2[k * qb:(k + 1) * qb, 32:64] for k in range(4)], axis=1)


def _node_upd_body(h1_ref, agga_ref, aggb_ref, n1a, n1b, n1bias, n2_W, n2_b,
                   out_W, out_b, hc_ref):
    h1 = h1_ref[...]
    agg = jnp.concatenate([agga_ref[0], aggb_ref[0]], axis=1)
    z = _silu(h1 @ n1a[...] + agg @ n1b[...] + n1bias[...])
    h1b = h1 + z @ n2_W[...] + n2_b[...]
    hc_ref[...] = h1b @ out_W[...] + out_b[...]


def _pool_head_body(hc_ref, b8_ref, m1_W, m1_b, m2_W, m2_b, m3_W, m3_b,
                    head_ref, acc_ref):
    i = pl.program_id(0)

    @pl.when(i == 0)
    def _():
        acc_ref[...] = jnp.zeros_like(acc_ref)

    bn = hc_ref.shape[0]
    oh = (b8_ref[:, 0:1] == lax.broadcasted_iota(I32, (bn, 512), 1)).astype(F32)
    acc_ref[...] += lax.dot_general(oh, hc_ref[...],
                                    (((0,), (0,)), ((), ())),
                                    preferred_element_type=F32)

    @pl.when(i == pl.num_programs(0) - 1)
    def _():
        pooled = acc_ref[...]
        z = jnp.maximum(pooled @ m1_W[...] + m1_b[...], 0.0)
        z = jnp.maximum(z @ m2_W[...] + m2_b[...], 0.0)
        head_ref[...] = z @ m3_W[...] + m3_b[...]


def _full(shape):
    return pl.BlockSpec(shape, lambda i: (0,) * len(shape))


def _blk(shape):
    return pl.BlockSpec(shape, lambda i: (i,) + (0,) * (len(shape) - 1))


def _make_gather(e_pad):
    per_w = e_pad // NW
    n_grp = per_w // GRP
    mesh = plsc.VectorSubcoreMesh(core_axis_name="c", subcore_axis_name="s")

    @functools.partial(
        pl.kernel, mesh=mesh,
        out_type=jax.ShapeDtypeStruct((e_pad, 128), F32),
        scratch_types=[pltpu.VMEM((8, 128), I32),
                       pltpu.VMEM((8, 128), I32),
                       pltpu.VMEM((CHG, 128), F32),
                       pltpu.VMEM((CHG, 128), F32),
                       pltpu.SemaphoreType.DMA,
                       pltpu.SemaphoreType.DMA],
    )
    def gather(tabA, tabB, idxA3d, idxB3d, outS,
               ia, ib, bufa, bufb, sema, semb):
        wid = lax.axis_index("s") * 2 + lax.axis_index("c")
        grp0 = wid * n_grp

        def body(i, carry):
            g = grp0 + i
            pltpu.sync_copy(idxA3d.at[g], ia)
            pltpu.sync_copy(idxB3d.at[g], ib)

            def sub_body(sub, carry2):
                ha = pltpu.async_copy(tabA.at[ia.at[sub]], bufa, sema)
                hb = pltpu.async_copy(tabB.at[ib.at[sub]], bufb, semb)
                ha.wait()
                hb.wait()
                for e in range(CHG):
                    for k in range(5):
                        sl = pl.ds(k * 16, 16)
                        bufa[e, sl] = bufa[e, sl] + bufb[e, sl]
                eb = pl.multiple_of(g * GRP + sub * CHG, CHG)
                pltpu.sync_copy(bufa, outS.at[pl.ds(eb, CHG)])
                return carry2

            lax.fori_loop(0, GRP // CHG, sub_body, 0)
            return carry

        lax.fori_loop(0, n_grp, body, 0)

    return gather


def _make_scatter(n_nodes, e_pad):
    n_acc = n_nodes + N_TRASH
    per_tile_rows = n_acc // 16
    drain_chunk = 3128
    per_t = e_pad // 16
    n_grp = per_t // GRP
    mesh = plsc.VectorSubcoreMesh(core_axis_name="c", subcore_axis_name="s")

    SUB = 128

    @functools.partial(
        pl.kernel, mesh=mesh,
        compiler_params=pltpu.CompilerParams(use_tc_tiling_on_sc=False),
        out_type=jax.ShapeDtypeStruct((2, n_nodes, 32), F32),
        scratch_types=[pltpu.VMEM((8, 128), I32),
                       pltpu.VMEM((SUB // 4, 128), F32),
                       pltpu.VMEM((SUB, 32), F32),
                       pltpu.VMEM_SHARED((n_acc, 32), F32)],
    )
    def scatter(m2a, m2b, idx3d, agg, idx, buf, upd, acc):
        c = lax.axis_index("c")
        s = lax.axis_index("s")

        z = jnp.zeros((16,), F32)
        for r in range(SUB):
            upd[r, pl.ds(0, 16)] = z
            upd[r, pl.ds(16, 16)] = z
        base = pl.multiple_of(s * per_tile_rows, SUB)
        for k in range(per_tile_rows // SUB):
            pltpu.sync_copy(upd, acc.at[pl.ds(base + k * SUB, SUB)])
        plsc.subcore_barrier()

        def run(m2ref):
            def body(i, carry):
                g = s * n_grp + i
                pltpu.sync_copy(idx3d.at[g], idx)
                def sub_body(sub, carry2):
                    mb = pl.multiple_of((g * GRP + sub * SUB) // 4, SUB // 4)
                    pltpu.sync_copy(m2ref.at[pl.ds(mb, SUB // 4)], buf)
                    for r in range(SUB // 4):
                        for q in range(4):
                            e = r * 4 + q
                            for k in range(2):
                                upd[e, pl.ds(k * 16, 16)] = (
                                    buf[r, pl.ds(q * 32 + k * 16, 16)])
                    pltpu.sync_copy(upd, acc.at[idx.at[sub]], add=True)
                    return carry2

                lax.fori_loop(0, GRP // SUB, sub_body, 0)
                return carry

            lax.fori_loop(0, n_grp, body, 0)

        @pl.when(c == 0)
        def _():
            run(m2a)

        @pl.when(c == 1)
        def _():
            run(m2b)

        plsc.subcore_barrier()
        dbase = pl.multiple_of(s * drain_chunk, 8)
        dlen_last = n_nodes - 15 * drain_chunk

        @pl.when(s < 15)
        def _():
            pltpu.sync_copy(acc.at[pl.ds(dbase, drain_chunk)],
                            agg.at[c, pl.ds(dbase, drain_chunk)])

        @pl.when(s == 15)
        def _():
            pltpu.sync_copy(acc.at[pl.ds(15 * drain_chunk, dlen_last)],
                            agg.at[c, pl.ds(15 * drain_chunk, dlen_last)])

    return scatter


def kernel(h, pe, x, t, context, edges, edge_index, edge_attr, batch, params):
    p = params
    N = h.shape[0]
    E = edge_attr.shape[0]
    G = 512
    BN = N // 10
    BE = 2048
    e_pad = ((E + NW * GRP - 1) // (NW * GRP)) * (NW * GRP)

    rows = edges[0]
    cols = edges[1]
    pad = e_pad - E
    rows_g = jnp.concatenate(
        [rows, jnp.zeros((pad,), I32)]).reshape(-1, 8, 128)
    cols_g = jnp.concatenate(
        [cols, jnp.zeros((pad,), I32)]).reshape(-1, 8, 128)
    rows_sp = jnp.concatenate(
        [rows, N + (jnp.arange(pad, dtype=I32) % 1024)])
    rows_s = rows_sp.reshape(-1, 4, 512).transpose(0, 2, 1).reshape(-1, 8, 128)
    ea_pad = jnp.concatenate([edge_attr, jnp.zeros((pad, 4), F32)])
    x_pad = jnp.pad(x, ((0, 0), (0, 13)))
    t8 = jnp.tile(t.reshape(N, 1), (1, 8))
    b8 = jnp.tile(batch.reshape(N, 1), (1, 8))

    convs = p['convs']
    folded = []
    for c in convs:
        folded.append(dict(
            Wi=c['e1_W'][0:64], Wj=c['e1_W'][64:128],
            wr=c['e1_W'][128:129],
            WeA=p['edge_W'] @ c['e1_W'][129:185],
            WeT=c['e1_W'][185:193],
            be=(c['e1_b'] + p['edge_b'] @ c['e1_W'][129:185]).reshape(1, 64),
            inW=c['in_W'], inb=c['in_b'].reshape(1, 64),
            n1a=c['n1_W'][:64], n1b=c['n1_W'][64:],
            n1bias=c['n1_b'].reshape(1, 64),
            n2W=c['n2_W'], n2b=c['n2_b'].reshape(1, 64),
            outW=c['out_W'], outb=c['out_b'].reshape(1, 64),
            e2W=c['e2_W'], e2b=c['e2_b'].reshape(1, 64)))

    stats = pl.pallas_call(
        _stats_body,
        grid=(1,),
        in_specs=[_full((N, 20))],
        out_specs=_full((8, 20)),
        out_shape=jax.ShapeDtypeStruct((8, 20), F32),
    )(pe)

    hc, tp1, tp2 = pl.pallas_call(
        _node_pre_body,
        grid=(10,),
        in_specs=[_blk((BN, 64)), _blk((BN, 20)), _blk((BN, 8)),
                  _blk((BN, 64)), _full((8, 20)),
                  _full((64, 32)), _full((1, 32)),
                  _full((20, 8)), _full((1, 8)),
                  _full((64, 16)), _full((1, 16)),
                  _full((1, 20)), _full((1, 20)),
                  _full((8, 64)), _full((8, 64))],
        out_specs=[_blk((BN, 64)), _blk((BN, 64)), _blk((BN, 64))],
        out_shape=[jax.ShapeDtypeStruct((N, 64), F32)] * 3,
    )(h, pe, t8, context, stats,
      p['node_W'], p['node_b'].reshape(1, 32),
      p['pe_W'], p['pe_b'].reshape(1, 8),
      p['ctx_W'], p['ctx_b'].reshape(1, 16),
      p['bn_w'].reshape(1, 20), p['bn_b'].reshape(1, 20),
      folded[0]['WeT'], folded[1]['WeT'])

    gather = _make_gather(e_pad)
    scatter = _make_scatter(N, e_pad)
    tps = [tp1, tp2]

    for li, fc in enumerate(folded):
        h1, ta, tb = pl.pallas_call(
            _node_lin_body,
            grid=(10,),
            in_specs=[_blk((BN, 64)), _blk((BN, 16)),
                      _full((64, 64)), _full((1, 64)),
                      _full((64, 64)), _full((64, 64))],
            out_specs=[_blk((BN, 64)), _blk((BN, 128)), _blk((BN, 128))],
            out_shape=[jax.ShapeDtypeStruct((N, 64), F32),
                       jax.ShapeDtypeStruct((N, 128), F32),
                       jax.ShapeDtypeStruct((N, 128), F32)],
        )(hc, x_pad, fc['inW'], fc['inb'], fc['Wi'], fc['Wj'])

        s_arr = gather(ta, tb, rows_g, cols_g)

        tp_full = jnp.concatenate(
            [tps[li], tps[li][0:1],
             jnp.zeros((e_pad // 16 - N - 1, 64), F32)])

        m2a, m2b = pl.pallas_call(
            _edge_mlp_body,
            grid=(e_pad // BE,),
            in_specs=[_blk((BE, 128)), _blk((BE, 4)),
                      _blk((BE // 16, 64)),
                      _full((1, 64)), _full((4, 64)), _full((1, 64)),
                      _full((64, 64)), _full((1, 64))],
            out_specs=[_blk((BE // 4, 128)), _blk((BE // 4, 128))],
            out_shape=[jax.ShapeDtypeStruct((e_pad // 4, 128), F32)] * 2,
        )(s_arr, ea_pad, tp_full,
          fc['wr'], fc['WeA'], fc['be'], fc['e2W'], fc['e2b'])

        aggs = scatter(m2a, m2b, rows_s)

        hc = pl.pallas_call(
            _node_upd_body,
            grid=(10,),
            in_specs=[_blk((BN, 64)),
                      pl.BlockSpec((1, BN, 32), lambda i: (0, i, 0)),
                      pl.BlockSpec((1, BN, 32), lambda i: (1, i, 0)),
                      _full((64, 64)), _full((64, 64)), _full((1, 64)),
                      _full((64, 64)), _full((1, 64)),
                      _full((64, 64)), _full((1, 64))],
            out_specs=_blk((BN, 64)),
            out_shape=jax.ShapeDtypeStruct((N, 64), F32),
        )(h1, aggs, aggs, fc['n1a'], fc['n1b'], fc['n1bias'],
          fc['n2W'], fc['n2b'], fc['outW'], fc['outb'])

    out = pl.pallas_call(
        _pool_head_body,
        grid=(10,),
        in_specs=[_blk((BN, 64)), _blk((BN, 8)),
                  _full((64, 32)), _full((1, 32)),
                  _full((32, 16)), _full((1, 16)),
                  _full((16, 1)), _full((1, 1))],
        out_specs=_full((G, 1)),
        out_shape=jax.ShapeDtypeStruct((G, 1), F32),
        scratch_shapes=[pltpu.VMEM((G, 64), F32)],
    )(hc, b8,
      p['m1_W'], p['m1_b'].reshape(1, 32),
      p['m2_W'], p['m2_b'].reshape(1, 16),
      p['m3_W'], p['m3_b'].reshape(1, 1))
    return out
